# baseline (speedup 1.0000x reference)
import functools

import jax
import jax.numpy as jnp
from jax import lax
from jax.experimental import pallas as pl
from jax.experimental.pallas import tpu as pltpu
from jax.experimental.pallas import tpu_sc as plsc

NC = 2
NS = 16
NW = NC * NS
LANES = 16
CH = 128
ANB = 2
ABS = 32
DNB = 2


def _mesh():
    return plsc.VectorSubcoreMesh(
        core_axis_name="c", subcore_axis_name="s", num_cores=NC, num_subcores=NS
    )


def _sc_aggregate(table2, src_r, dst_r, npad, with_count):
    dh = table2.shape[2]
    nch = src_r.shape[1]
    nb = nch // ABS
    dl = dh // LANES
    rows_pt = npad // NS
    nz = rows_pt // CH

    out_types = [jax.ShapeDtypeStruct((NC, npad, dh), jnp.float32)]
    if with_count:
        out_types.append(jax.ShapeDtypeStruct((NC, npad, LANES), jnp.float32))
    scratch = [
        pltpu.VMEM((2, ABS, CH), jnp.int32),
        pltpu.VMEM((2, ABS, CH), jnp.int32),
        pltpu.VMEM((CH, LANES), jnp.float32),
        pltpu.VMEM_SHARED((npad, dh), jnp.float32),
        pltpu.VMEM_SHARED((npad, dh), jnp.float32),
        pltpu.VMEM_SHARED((npad, LANES), jnp.float32),
        pltpu.SemaphoreType.DMA,
    ]
    scratch += [pltpu.VMEM((CH, dh), jnp.float32) for _ in range(ANB)]
    scratch += [pltpu.SemaphoreType.DMA] * (3 * ANB)

    @functools.partial(
        pl.kernel,
        out_type=tuple(out_types),
        mesh=_mesh(),
        scratch_types=scratch,
        compiler_params=pltpu.CompilerParams(use_tc_tiling_on_sc=False),
    )
    def run(table_h, src_h, dst_h, *rest):
        if with_count:
            agg_o, cnt_o, idx_s, idx_d, ones, tab_sh, acc_sh, cnt_sh, isem = rest[:9]
            rest = rest[9:]
        else:
            agg_o, idx_s, idx_d, ones, tab_sh, acc_sh, cnt_sh, isem = rest[:8]
            rest = rest[8:]
        bufs = rest[:ANB]
        gsem = rest[ANB:2 * ANB]
        ssem = rest[2 * ANB:3 * ANB]
        csem = rest[3 * ANB:4 * ANB]
        c = lax.axis_index("c")
        s = lax.axis_index("s")
        base = s * rows_pt

        zero = jnp.zeros((LANES,), jnp.float32)
        buf0 = bufs[0]

        def zrow(i, _):
            for j in range(dl):
                buf0[i, pl.ds(j * LANES, LANES)] = zero
            ones[i, pl.ds(0, LANES)] = zero
            return 0

        lax.fori_loop(0, CH, zrow, 0)
        for k in range(nz):
            pltpu.sync_copy(buf0, acc_sh.at[pl.ds(base + k * CH, CH)])
        if with_count:
            for k in range(nz):
                pltpu.sync_copy(ones, cnt_sh.at[pl.ds(base + k * CH, CH)])

            one = jnp.full((LANES,), 1.0, jnp.float32)

            def orow(i, _):
                ones[i, pl.ds(0, LANES)] = one
                return 0

            lax.fori_loop(0, CH, orow, 0)

        pltpu.sync_copy(
            table_h.at[c, pl.ds(base, rows_pt)], tab_sh.at[pl.ds(base, rows_pt)]
        )
        pltpu.sync_copy(src_h.at[s, pl.ds(0, ABS)], idx_s.at[0])
        pltpu.sync_copy(dst_h.at[s, pl.ds(0, ABS)], idx_d.at[0])
        plsc.subcore_barrier()

        def batch(bi, _):
            sl = lax.rem(bi, 2)
            ia = idx_s.at[sl]
            idd = idx_d.at[sl]
            count_here = lax.rem(bi, 2) == c

            @pl.when(bi + 1 < nb)
            def _():
                sq = lax.rem(bi + 1, 2)
                pltpu.async_copy(
                    src_h.at[s, pl.ds((bi + 1) * ABS, ABS)], idx_s.at[sq], isem
                )
                pltpu.async_copy(
                    dst_h.at[s, pl.ds((bi + 1) * ABS, ABS)], idx_d.at[sq], isem
                )

            def gather(j, b):
                pltpu.async_copy(tab_sh.at[ia.at[j]], bufs[b], gsem[b])

            def gather_wait(j, b):
                pltpu.make_async_copy(tab_sh.at[ia.at[j]], bufs[b], gsem[b]).wait()

            def scat(j, b):
                pltpu.async_copy(bufs[b], acc_sh.at[idd.at[j]], ssem[b], add=True)

            def scat_wait(j, b):
                pltpu.make_async_copy(bufs[b], acc_sh.at[idd.at[j]], ssem[b]).wait()

            def cnt_scat(j, b):
                pltpu.async_copy(ones, cnt_sh.at[idd.at[j]], csem[b], add=True)

            def cnt_wait(j, b):
                pltpu.make_async_copy(ones, cnt_sh.at[idd.at[j]], csem[b]).wait()

            for b in range(ANB - 1):
                gather(b, b)
            for b in range(ANB):
                if b == 0:
                    gather(ANB - 1, ANB - 1)
                gather_wait(b, b)
                scat(b, b)
                if with_count:
                    @pl.when(count_here)
                    def _():
                        cnt_scat(b, b)
                if b >= 1:
                    scat_wait(b - 1, b - 1)
                    gather(b + ANB - 1, b - 1)

            def group(g, _):
                for b in range(ANB):
                    j = g * ANB + b
                    gather_wait(j, b)
                    if with_count:
                        @pl.when(count_here)
                        def _():
                            cnt_wait(j - ANB, b)
                            cnt_scat(j, b)
                    scat(j, b)
                    bn = (b - 1) % ANB
                    scat_wait(j - 1, bn)

                    @pl.when(j + ANB - 1 < ABS)
                    def _():
                        gather(j + ANB - 1, bn)
                return 0

            lax.fori_loop(1, ABS // ANB, group, 0)
            scat_wait(ABS - 1, (ABS - 1) % ANB)
            if with_count:
                @pl.when(count_here)
                def _():
                    for b in range(ANB):
                        cnt_wait(ABS - ANB + b, (ABS - ANB + b) % ANB)

            @pl.when(bi + 1 < nb)
            def _():
                sq = lax.rem(bi + 1, 2)
                pltpu.make_async_copy(
                    src_h.at[s, pl.ds((bi + 1) * ABS, ABS)], idx_s.at[sq], isem
                ).wait()
                pltpu.make_async_copy(
                    dst_h.at[s, pl.ds((bi + 1) * ABS, ABS)], idx_d.at[sq], isem
                ).wait()
            return 0

        lax.fori_loop(0, nb, batch, 0)
        plsc.subcore_barrier()
        pltpu.sync_copy(
            acc_sh.at[pl.ds(base, rows_pt)], agg_o.at[c, pl.ds(base, rows_pt)]
        )
        if with_count:
            pltpu.sync_copy(
                cnt_sh.at[pl.ds(base, rows_pt)], cnt_o.at[c, pl.ds(base, rows_pt)]
            )

    res = run(table2, src_r, dst_r)
    if with_count:
        return res[0], res[1]
    return res[0] if isinstance(res, (tuple, list)) else res


DCH = 64


def _sc_decode(z2, la_r, lb_r, npad):
    hk = z2.shape[2]
    nch = la_r.shape[1]
    gpc = DCH // LANES
    rows_pt = npad // NS

    scratch = [
        pltpu.VMEM((nch, DCH), jnp.int32),
        pltpu.VMEM((nch, DCH), jnp.int32),
        pltpu.VMEM((nch, gpc, LANES), jnp.float32),
        pltpu.VMEM_SHARED((npad, hk, LANES), jnp.float32),
    ]
    scratch += [pltpu.VMEM((DCH, hk, LANES), jnp.float32) for _ in range(2 * DNB)]
    scratch += [pltpu.SemaphoreType.DMA] * (2 * DNB)

    @functools.partial(
        pl.kernel,
        out_type=jax.ShapeDtypeStruct((NC, NS, nch, gpc, LANES), jnp.float32),
        mesh=_mesh(),
        scratch_types=scratch,
        compiler_params=pltpu.CompilerParams(
            use_tc_tiling_on_sc=False, needs_layout_passes=False
        ),
    )
    def run(z_h, la_h, lb_h, out_h, idx_a, idx_b, obuf, z_sh, *rest):
        bufa = rest[:DNB]
        bufb = rest[DNB:2 * DNB]
        sema = rest[2 * DNB:3 * DNB]
        semb = rest[3 * DNB:4 * DNB]
        c = lax.axis_index("c")
        s = lax.axis_index("s")
        lanes = lax.iota(jnp.int32, LANES)

        base = s * rows_pt
        pltpu.sync_copy(
            z_h.at[c, pl.ds(base, rows_pt)], z_sh.at[pl.ds(base, rows_pt)]
        )
        pltpu.sync_copy(la_h.at[s], idx_a)
        pltpu.sync_copy(lb_h.at[s], idx_b)
        plsc.subcore_barrier()

        def fire(j, b):
            pltpu.async_copy(z_sh.at[idx_a.at[j]], bufa[b], sema[b])
            pltpu.async_copy(z_sh.at[idx_b.at[j]], bufb[b], semb[b])

        def wait(j, b):
            pltpu.make_async_copy(z_sh.at[idx_a.at[j]], bufa[b], sema[b]).wait()
            pltpu.make_async_copy(z_sh.at[idx_b.at[j]], bufb[b], semb[b]).wait()

        for b in range(DNB):
            fire(b, b)

        def group(g, _):
            for b in range(DNB):
                j = g * DNB + b
                wait(j, b)

                def grp16(t, _):
                    e0 = t * LANES
                    parts = []
                    for ee in range(LANES):
                        acc = bufa[b][e0 + ee, 0] * bufb[b][e0 + ee, 0]
                        for k in range(1, hk):
                            acc = acc + bufa[b][e0 + ee, k] * bufb[b][e0 + ee, k]
                        parts.append(jnp.where(lanes == ee, jnp.sum(acc), 0.0))
                    while len(parts) > 1:
                        parts = [
                            parts[i] + parts[i + 1]
                            for i in range(0, len(parts), 2)
                        ]
                    obuf[j, t] = parts[0]
                    return 0

                lax.fori_loop(0, gpc, grp16, 0)

                @pl.when(j + DNB < nch)
                def _():
                    fire(j + DNB, b)
            return 0

        lax.fori_loop(0, nch // DNB, group, 0)
        pltpu.sync_copy(obuf, out_h.at[c, s])

    return run(z2, la_r, lb_r)


def _tc_pair_sum(p):
    _, m, d = p.shape

    def body(p_r, o_r):
        o_r[...] = p_r[0] + p_r[1]

    return pl.pallas_call(
        body,
        out_shape=jax.ShapeDtypeStruct((m, d), jnp.float32),
    )(p)


def _tc_layer(agg, cnt, xin, wl, wr, b, relu, split_out):
    npad = agg.shape[1]
    dh = agg.shape[2]
    d = 2 * dh
    br = 1024
    grid = (npad // br,)
    b2 = b.reshape(1, d)

    def body(a_r, cnt_r, x_r, wl_r, wr_r, b_r, out_r):
        aggf = jnp.concatenate([a_r[0], a_r[1]], axis=1)
        xf = jnp.concatenate([x_r[0], x_r[1]], axis=1)
        cntc = jnp.clip(cnt_r[0][:, 0:1] + cnt_r[1][:, 0:1], 1.0, None)
        r = (
            jnp.dot(aggf / cntc, wl_r[...], preferred_element_type=jnp.float32)
            + jnp.dot(xf, wr_r[...], preferred_element_type=jnp.float32)
            + b_r[...]
        )
        if relu:
            r = jnp.maximum(r, 0.0)
        if split_out:
            out_r[0] = r[:, :dh]
            out_r[1] = r[:, dh:]
        else:
            out_r[...] = r

    if split_out:
        out_spec = pl.BlockSpec((NC, br, dh), lambda i: (0, i, 0))
        out_shape = jax.ShapeDtypeStruct((NC, npad, dh), jnp.float32)
    else:
        out_spec = pl.BlockSpec((br, d), lambda i: (i, 0))
        out_shape = jax.ShapeDtypeStruct((npad, d), jnp.float32)

    return pl.pallas_call(
        body,
        grid=grid,
        in_specs=[
            pl.BlockSpec((NC, br, dh), lambda i: (0, i, 0)),
            pl.BlockSpec((NC, br, LANES), lambda i: (0, i, 0)),
            pl.BlockSpec((NC, br, dh), lambda i: (0, i, 0)),
            pl.BlockSpec((d, d), lambda i: (0, 0)),
            pl.BlockSpec((d, d), lambda i: (0, 0)),
            pl.BlockSpec((1, d), lambda i: (0, 0)),
        ],
        out_specs=out_spec,
        out_shape=out_shape,
    )(agg, cnt, xin, wl, wr, b2)


def _tc_split(xin):
    npad, d = xin.shape
    dh = d // 2
    br = 1024

    def body(x_r, o_r):
        o_r[0] = x_r[:, :dh]
        o_r[1] = x_r[:, dh:]

    return pl.pallas_call(
        body,
        grid=(npad // br,),
        in_specs=[pl.BlockSpec((br, d), lambda i: (i, 0))],
        out_specs=pl.BlockSpec((NC, br, dh), lambda i: (0, i, 0)),
        out_shape=jax.ShapeDtypeStruct((NC, npad, dh), jnp.float32),
    )(xin)


def kernel(x, edge_index, edge_label_index, W1l, W1r, b1, W2l, W2r, b2):
    n, d = x.shape
    e = edge_index.shape[1]
    dh = d // 2

    npad = -(-n // 256) * 256
    if npad == n:
        npad += 256
    epa = -(-e // (NS * CH * ABS)) * (NS * CH * ABS)
    nca = epa // (NS * CH)
    epd = -(-e // (NS * DCH * DNB)) * (NS * DCH * DNB)
    ncd = epd // (NS * DCH)

    src = jnp.pad(edge_index[0], (0, epa - e)).reshape(NS, nca, CH)
    dst = jnp.pad(edge_index[1], (0, epa - e), constant_values=n).reshape(NS, nca, CH)

    la = jnp.pad(edge_label_index[0], (0, epd - e)).reshape(NS, ncd, DCH)
    lb = jnp.pad(edge_label_index[1], (0, epd - e)).reshape(NS, ncd, DCH)

    xp = jnp.pad(x, ((0, npad - n), (0, 0)))
    xsplit = _tc_split(xp)

    agg1, cnt = _sc_aggregate(xsplit, src, dst, npad, with_count=True)
    hsplit = _tc_layer(agg1, cnt, xsplit, W1l, W1r, b1, relu=True, split_out=True)
    agg2 = _sc_aggregate(hsplit, src, dst, npad, with_count=False)
    zsplit = _tc_layer(agg2, cnt, hsplit, W2l, W2r, b2, relu=False, split_out=True)
    z2 = zsplit.reshape(NC, npad, dh // LANES, LANES)
    part = _sc_decode(z2, la, lb, npad)
    p = part.reshape(NC, (NS * ncd * DCH) // d, d)
    out = _tc_pair_sum(p)
    return out.reshape(-1)[:e]


# speedup vs baseline: 1.0212x; 1.0212x over previous
import functools

import jax
import jax.numpy as jnp
from jax import lax
from jax.experimental import pallas as pl
from jax.experimental.pallas import tpu as pltpu
from jax.experimental.pallas import tpu_sc as plsc

NC = 2
NS = 16
NW = NC * NS
LANES = 16
CH = 128
ANB = 2
ABS = 32
DNB = 2


def _mesh():
    return plsc.VectorSubcoreMesh(
        core_axis_name="c", subcore_axis_name="s", num_cores=NC, num_subcores=NS
    )


def _sc_aggregate(table2, src_r, dst_r, npad, with_count):
    dh = table2.shape[2]
    nch = src_r.shape[1]
    nb = nch // ABS
    dl = dh // LANES
    rows_pt = npad // NS
    nz = rows_pt // CH

    out_types = [jax.ShapeDtypeStruct((NC, npad, dh), jnp.float32)]
    if with_count:
        out_types.append(jax.ShapeDtypeStruct((NC, npad, LANES), jnp.float32))
    scratch = [
        pltpu.VMEM((2, ABS, CH), jnp.int32),
        pltpu.VMEM((2, ABS, CH), jnp.int32),
        pltpu.VMEM((CH, LANES), jnp.float32),
        pltpu.VMEM_SHARED((npad, dh), jnp.float32),
        pltpu.VMEM_SHARED((npad, dh), jnp.float32),
        pltpu.VMEM_SHARED((npad, LANES), jnp.float32),
        pltpu.SemaphoreType.DMA,
    ]
    scratch += [pltpu.VMEM((CH, dh), jnp.float32) for _ in range(ANB)]
    scratch += [pltpu.SemaphoreType.DMA] * (3 * ANB)

    @functools.partial(
        pl.kernel,
        out_type=tuple(out_types),
        mesh=_mesh(),
        scratch_types=scratch,
        compiler_params=pltpu.CompilerParams(use_tc_tiling_on_sc=False),
    )
    def run(table_h, src_h, dst_h, *rest):
        if with_count:
            agg_o, cnt_o, idx_s, idx_d, ones, tab_sh, acc_sh, cnt_sh, isem = rest[:9]
            rest = rest[9:]
        else:
            agg_o, idx_s, idx_d, ones, tab_sh, acc_sh, cnt_sh, isem = rest[:8]
            rest = rest[8:]
        bufs = rest[:ANB]
        gsem = rest[ANB:2 * ANB]
        ssem = rest[2 * ANB:3 * ANB]
        csem = rest[3 * ANB:4 * ANB]
        c = lax.axis_index("c")
        s = lax.axis_index("s")
        base = s * rows_pt

        zero = jnp.zeros((LANES,), jnp.float32)
        buf0 = bufs[0]

        def zrow(i, _):
            for j in range(dl):
                buf0[i, pl.ds(j * LANES, LANES)] = zero
            ones[i, pl.ds(0, LANES)] = zero
            return 0

        lax.fori_loop(0, CH, zrow, 0)
        for k in range(nz):
            pltpu.sync_copy(buf0, acc_sh.at[pl.ds(base + k * CH, CH)])
        if with_count:
            for k in range(nz):
                pltpu.sync_copy(ones, cnt_sh.at[pl.ds(base + k * CH, CH)])

            one = jnp.full((LANES,), 1.0, jnp.float32)

            def orow(i, _):
                ones[i, pl.ds(0, LANES)] = one
                return 0

            lax.fori_loop(0, CH, orow, 0)

        pltpu.sync_copy(
            table_h.at[c, pl.ds(base, rows_pt)], tab_sh.at[pl.ds(base, rows_pt)]
        )
        pltpu.sync_copy(src_h.at[s, pl.ds(0, ABS)], idx_s.at[0])
        pltpu.sync_copy(dst_h.at[s, pl.ds(0, ABS)], idx_d.at[0])
        plsc.subcore_barrier()

        def batch(bi, _):
            sl = lax.rem(bi, 2)
            ia = idx_s.at[sl]
            idd = idx_d.at[sl]
            count_here = lax.rem(bi, 2) == c

            @pl.when(bi + 1 < nb)
            def _():
                sq = lax.rem(bi + 1, 2)
                pltpu.async_copy(
                    src_h.at[s, pl.ds((bi + 1) * ABS, ABS)], idx_s.at[sq], isem
                )
                pltpu.async_copy(
                    dst_h.at[s, pl.ds((bi + 1) * ABS, ABS)], idx_d.at[sq], isem
                )

            def gather(j, b):
                pltpu.async_copy(tab_sh.at[ia.at[j]], bufs[b], gsem[b])

            def gather_wait(j, b):
                pltpu.make_async_copy(tab_sh.at[ia.at[j]], bufs[b], gsem[b]).wait()

            def scat(j, b):
                pltpu.async_copy(bufs[b], acc_sh.at[idd.at[j]], ssem[b], add=True)

            def scat_wait(j, b):
                pltpu.make_async_copy(bufs[b], acc_sh.at[idd.at[j]], ssem[b]).wait()

            def cnt_scat(j, b):
                pltpu.async_copy(ones, cnt_sh.at[idd.at[j]], csem[b], add=True)

            def cnt_wait(j, b):
                pltpu.make_async_copy(ones, cnt_sh.at[idd.at[j]], csem[b]).wait()

            for b in range(ANB - 1):
                gather(b, b)
            for b in range(ANB):
                if b == 0:
                    gather(ANB - 1, ANB - 1)
                gather_wait(b, b)
                scat(b, b)
                if with_count:
                    @pl.when(count_here)
                    def _():
                        cnt_scat(b, b)
                if b >= 1:
                    scat_wait(b - 1, b - 1)
                    gather(b + ANB - 1, b - 1)

            def group(g, _):
                for b in range(ANB):
                    j = g * ANB + b
                    gather_wait(j, b)
                    if with_count:
                        @pl.when(count_here)
                        def _():
                            cnt_wait(j - ANB, b)
                            cnt_scat(j, b)
                    scat(j, b)
                    bn = (b - 1) % ANB
                    scat_wait(j - 1, bn)

                    @pl.when(j + ANB - 1 < ABS)
                    def _():
                        gather(j + ANB - 1, bn)
                return 0

            lax.fori_loop(1, ABS // ANB, group, 0)
            scat_wait(ABS - 1, (ABS - 1) % ANB)
            if with_count:
                @pl.when(count_here)
                def _():
                    for b in range(ANB):
                        cnt_wait(ABS - ANB + b, (ABS - ANB + b) % ANB)

            @pl.when(bi + 1 < nb)
            def _():
                sq = lax.rem(bi + 1, 2)
                pltpu.make_async_copy(
                    src_h.at[s, pl.ds((bi + 1) * ABS, ABS)], idx_s.at[sq], isem
                ).wait()
                pltpu.make_async_copy(
                    dst_h.at[s, pl.ds((bi + 1) * ABS, ABS)], idx_d.at[sq], isem
                ).wait()
            return 0

        lax.fori_loop(0, nb, batch, 0)
        plsc.subcore_barrier()
        pltpu.sync_copy(
            acc_sh.at[pl.ds(base, rows_pt)], agg_o.at[c, pl.ds(base, rows_pt)]
        )
        if with_count:
            pltpu.sync_copy(
                cnt_sh.at[pl.ds(base, rows_pt)], cnt_o.at[c, pl.ds(base, rows_pt)]
            )

    res = run(table2, src_r, dst_r)
    if with_count:
        return res[0], res[1]
    return res[0] if isinstance(res, (tuple, list)) else res


DCH = 64


def _sc_decode(z2, la_r, lb_r, npad):
    hk = z2.shape[2]
    nch = la_r.shape[1]
    gpc = DCH // LANES
    rows_pt = npad // NS

    scratch = [
        pltpu.VMEM((nch, DCH), jnp.int32),
        pltpu.VMEM((nch, DCH), jnp.int32),
        pltpu.VMEM((nch, gpc, LANES), jnp.float32),
        pltpu.VMEM_SHARED((npad, hk, LANES), jnp.float32),
    ]
    scratch += [pltpu.VMEM((DCH, hk, LANES), jnp.float32) for _ in range(2 * DNB)]
    scratch += [pltpu.SemaphoreType.DMA] * (2 * DNB)

    @functools.partial(
        pl.kernel,
        out_type=jax.ShapeDtypeStruct((NC, NS, nch, gpc, LANES), jnp.float32),
        mesh=_mesh(),
        scratch_types=scratch,
        compiler_params=pltpu.CompilerParams(
            use_tc_tiling_on_sc=False, needs_layout_passes=False
        ),
    )
    def run(z_h, la_h, lb_h, out_h, idx_a, idx_b, obuf, z_sh, *rest):
        bufa = rest[:DNB]
        bufb = rest[DNB:2 * DNB]
        sema = rest[2 * DNB:3 * DNB]
        semb = rest[3 * DNB:4 * DNB]
        c = lax.axis_index("c")
        s = lax.axis_index("s")
        lanes = lax.iota(jnp.int32, LANES)

        base = s * rows_pt
        pltpu.sync_copy(
            z_h.at[c, pl.ds(base, rows_pt)], z_sh.at[pl.ds(base, rows_pt)]
        )
        pltpu.sync_copy(la_h.at[s], idx_a)
        pltpu.sync_copy(lb_h.at[s], idx_b)
        plsc.subcore_barrier()

        def fire(j, b):
            pltpu.async_copy(z_sh.at[idx_a.at[j]], bufa[b], sema[b])
            pltpu.async_copy(z_sh.at[idx_b.at[j]], bufb[b], semb[b])

        def wait(j, b):
            pltpu.make_async_copy(z_sh.at[idx_a.at[j]], bufa[b], sema[b]).wait()
            pltpu.make_async_copy(z_sh.at[idx_b.at[j]], bufb[b], semb[b]).wait()

        for b in range(DNB):
            fire(b, b)

        def group(g, _):
            for b in range(DNB):
                j = g * DNB + b
                wait(j, b)

                def grp16(t, _):
                    e0 = t * LANES
                    vec = jnp.zeros((LANES,), jnp.float32)
                    for ee in range(LANES):
                        acc = bufa[b][e0 + ee, 0] * bufb[b][e0 + ee, 0]
                        for k in range(1, hk):
                            acc = acc + bufa[b][e0 + ee, k] * bufb[b][e0 + ee, k]
                        vec = jnp.where(lanes == ee, jnp.sum(acc), vec)
                    obuf[j, t] = vec
                    return 0

                lax.fori_loop(0, gpc, grp16, 0)

                @pl.when(j + DNB < nch)
                def _():
                    fire(j + DNB, b)
            return 0

        lax.fori_loop(0, nch // DNB, group, 0)
        pltpu.sync_copy(obuf, out_h.at[c, s])

    return run(z2, la_r, lb_r)


def _tc_pair_sum(p):
    _, m, d = p.shape

    def body(p_r, o_r):
        o_r[...] = p_r[0] + p_r[1]

    return pl.pallas_call(
        body,
        out_shape=jax.ShapeDtypeStruct((m, d), jnp.float32),
    )(p)


def _tc_layer(agg, cnt, xin, wl, wr, b, relu, split_out):
    npad = agg.shape[1]
    dh = agg.shape[2]
    d = 2 * dh
    br = 1024
    grid = (npad // br,)
    b2 = b.reshape(1, d)

    def body(a_r, cnt_r, x_r, wl_r, wr_r, b_r, out_r):
        aggf = jnp.concatenate([a_r[0], a_r[1]], axis=1)
        xf = jnp.concatenate([x_r[0], x_r[1]], axis=1)
        cntc = jnp.clip(cnt_r[0][:, 0:1] + cnt_r[1][:, 0:1], 1.0, None)
        r = (
            jnp.dot(aggf / cntc, wl_r[...], preferred_element_type=jnp.float32)
            + jnp.dot(xf, wr_r[...], preferred_element_type=jnp.float32)
            + b_r[...]
        )
        if relu:
            r = jnp.maximum(r, 0.0)
        if split_out:
            out_r[0] = r[:, :dh]
            out_r[1] = r[:, dh:]
        else:
            out_r[...] = r

    if split_out:
        out_spec = pl.BlockSpec((NC, br, dh), lambda i: (0, i, 0))
        out_shape = jax.ShapeDtypeStruct((NC, npad, dh), jnp.float32)
    else:
        out_spec = pl.BlockSpec((br, d), lambda i: (i, 0))
        out_shape = jax.ShapeDtypeStruct((npad, d), jnp.float32)

    return pl.pallas_call(
        body,
        grid=grid,
        in_specs=[
            pl.BlockSpec((NC, br, dh), lambda i: (0, i, 0)),
            pl.BlockSpec((NC, br, LANES), lambda i: (0, i, 0)),
            pl.BlockSpec((NC, br, dh), lambda i: (0, i, 0)),
            pl.BlockSpec((d, d), lambda i: (0, 0)),
            pl.BlockSpec((d, d), lambda i: (0, 0)),
            pl.BlockSpec((1, d), lambda i: (0, 0)),
        ],
        out_specs=out_spec,
        out_shape=out_shape,
    )(agg, cnt, xin, wl, wr, b2)


def _tc_split(xin):
    npad, d = xin.shape
    dh = d // 2
    br = 1024

    def body(x_r, o_r):
        o_r[0] = x_r[:, :dh]
        o_r[1] = x_r[:, dh:]

    return pl.pallas_call(
        body,
        grid=(npad // br,),
        in_specs=[pl.BlockSpec((br, d), lambda i: (i, 0))],
        out_specs=pl.BlockSpec((NC, br, dh), lambda i: (0, i, 0)),
        out_shape=jax.ShapeDtypeStruct((NC, npad, dh), jnp.float32),
    )(xin)


def kernel(x, edge_index, edge_label_index, W1l, W1r, b1, W2l, W2r, b2):
    n, d = x.shape
    e = edge_index.shape[1]
    dh = d // 2

    npad = -(-n // 256) * 256
    if npad == n:
        npad += 256
    epa = -(-e // (NS * CH * ABS)) * (NS * CH * ABS)
    nca = epa // (NS * CH)
    epd = -(-e // (NS * DCH * DNB)) * (NS * DCH * DNB)
    ncd = epd // (NS * DCH)

    src = jnp.pad(edge_index[0], (0, epa - e)).reshape(NS, nca, CH)
    dst = jnp.pad(edge_index[1], (0, epa - e), constant_values=n).reshape(NS, nca, CH)

    la = jnp.pad(edge_label_index[0], (0, epd - e)).reshape(NS, ncd, DCH)
    lb = jnp.pad(edge_label_index[1], (0, epd - e)).reshape(NS, ncd, DCH)

    xp = jnp.pad(x, ((0, npad - n), (0, 0)))
    xsplit = _tc_split(xp)

    agg1, cnt = _sc_aggregate(xsplit, src, dst, npad, with_count=True)
    hsplit = _tc_layer(agg1, cnt, xsplit, W1l, W1r, b1, relu=True, split_out=True)
    agg2 = _sc_aggregate(hsplit, src, dst, npad, with_count=False)
    zsplit = _tc_layer(agg2, cnt, hsplit, W2l, W2r, b2, relu=False, split_out=True)
    z2 = zsplit.reshape(NC, npad, dh // LANES, LANES)
    part = _sc_decode(z2, la, lb, npad)
    p = part.reshape(NC, (NS * ncd * DCH) // d, d)
    out = _tc_pair_sum(p)
    return out.reshape(-1)[:e]


# speedup vs baseline: 1.0331x; 1.0116x over previous
import functools

import jax
import jax.numpy as jnp
from jax import lax
from jax.experimental import pallas as pl
from jax.experimental.pallas import tpu as pltpu
from jax.experimental.pallas import tpu_sc as plsc

NC = 2
NS = 16
NW = NC * NS
LANES = 16
CH = 128
ANB = 2
ABS = 40
DNB = 2


def _mesh():
    return plsc.VectorSubcoreMesh(
        core_axis_name="c", subcore_axis_name="s", num_cores=NC, num_subcores=NS
    )


def _sc_aggregate(table2, src_r, dst_r, npad, with_count):
    dh = table2.shape[2]
    nch = src_r.shape[1]
    nb = nch // ABS
    dl = dh // LANES
    rows_pt = npad // NS
    nz = rows_pt // CH

    out_types = [jax.ShapeDtypeStruct((NC, npad, dh), jnp.float32)]
    if with_count:
        out_types.append(jax.ShapeDtypeStruct((NC, npad, LANES), jnp.float32))
    scratch = [
        pltpu.VMEM((2, ABS, CH), jnp.int32),
        pltpu.VMEM((2, ABS, CH), jnp.int32),
        pltpu.VMEM((CH, LANES), jnp.float32),
        pltpu.VMEM_SHARED((npad, dh), jnp.float32),
        pltpu.VMEM_SHARED((npad, dh), jnp.float32),
        pltpu.VMEM_SHARED((npad, LANES), jnp.float32),
        pltpu.SemaphoreType.DMA,
    ]
    scratch += [pltpu.VMEM((CH, dh), jnp.float32) for _ in range(ANB)]
    scratch += [pltpu.SemaphoreType.DMA] * (3 * ANB)

    @functools.partial(
        pl.kernel,
        out_type=tuple(out_types),
        mesh=_mesh(),
        scratch_types=scratch,
        compiler_params=pltpu.CompilerParams(use_tc_tiling_on_sc=False),
    )
    def run(table_h, src_h, dst_h, *rest):
        if with_count:
            agg_o, cnt_o, idx_s, idx_d, ones, tab_sh, acc_sh, cnt_sh, isem = rest[:9]
            rest = rest[9:]
        else:
            agg_o, idx_s, idx_d, ones, tab_sh, acc_sh, cnt_sh, isem = rest[:8]
            rest = rest[8:]
        bufs = rest[:ANB]
        gsem = rest[ANB:2 * ANB]
        ssem = rest[2 * ANB:3 * ANB]
        csem = rest[3 * ANB:4 * ANB]
        c = lax.axis_index("c")
        s = lax.axis_index("s")
        base = s * rows_pt

        zero = jnp.zeros((LANES,), jnp.float32)
        buf0 = bufs[0]

        def zrow(i, _):
            for j in range(dl):
                buf0[i, pl.ds(j * LANES, LANES)] = zero
            ones[i, pl.ds(0, LANES)] = zero
            return 0

        lax.fori_loop(0, CH, zrow, 0)
        for k in range(nz):
            pltpu.sync_copy(buf0, acc_sh.at[pl.ds(base + k * CH, CH)])
        if with_count:
            for k in range(nz):
                pltpu.sync_copy(ones, cnt_sh.at[pl.ds(base + k * CH, CH)])

            one = jnp.full((LANES,), 1.0, jnp.float32)

            def orow(i, _):
                ones[i, pl.ds(0, LANES)] = one
                return 0

            lax.fori_loop(0, CH, orow, 0)

        pltpu.sync_copy(
            table_h.at[c, pl.ds(base, rows_pt)], tab_sh.at[pl.ds(base, rows_pt)]
        )
        pltpu.sync_copy(src_h.at[s, pl.ds(0, ABS)], idx_s.at[0])
        pltpu.sync_copy(dst_h.at[s, pl.ds(0, ABS)], idx_d.at[0])
        plsc.subcore_barrier()

        def batch(bi, _):
            sl = lax.rem(bi, 2)
            ia = idx_s.at[sl]
            idd = idx_d.at[sl]
            count_here = lax.rem(bi, 2) == c

            @pl.when(bi + 1 < nb)
            def _():
                sq = lax.rem(bi + 1, 2)
                pltpu.async_copy(
                    src_h.at[s, pl.ds((bi + 1) * ABS, ABS)], idx_s.at[sq], isem
                )
                pltpu.async_copy(
                    dst_h.at[s, pl.ds((bi + 1) * ABS, ABS)], idx_d.at[sq], isem
                )

            def gather(j, b):
                pltpu.async_copy(tab_sh.at[ia.at[j]], bufs[b], gsem[b])

            def gather_wait(j, b):
                pltpu.make_async_copy(tab_sh.at[ia.at[j]], bufs[b], gsem[b]).wait()

            def scat(j, b):
                pltpu.async_copy(bufs[b], acc_sh.at[idd.at[j]], ssem[b], add=True)

            def scat_wait(j, b):
                pltpu.make_async_copy(bufs[b], acc_sh.at[idd.at[j]], ssem[b]).wait()

            def cnt_scat(j, b):
                pltpu.async_copy(ones, cnt_sh.at[idd.at[j]], csem[b], add=True)

            def cnt_wait(j, b):
                pltpu.make_async_copy(ones, cnt_sh.at[idd.at[j]], csem[b]).wait()

            for b in range(ANB - 1):
                gather(b, b)
            for b in range(ANB):
                if b == 0:
                    gather(ANB - 1, ANB - 1)
                gather_wait(b, b)
                scat(b, b)
                if with_count:
                    @pl.when(count_here)
                    def _():
                        cnt_scat(b, b)
                if b >= 1:
                    scat_wait(b - 1, b - 1)
                    gather(b + ANB - 1, b - 1)

            def group(g, _):
                for b in range(ANB):
                    j = g * ANB + b
                    gather_wait(j, b)
                    if with_count:
                        @pl.when(count_here)
                        def _():
                            cnt_wait(j - ANB, b)
                            cnt_scat(j, b)
                    scat(j, b)
                    bn = (b - 1) % ANB
                    scat_wait(j - 1, bn)

                    @pl.when(j + ANB - 1 < ABS)
                    def _():
                        gather(j + ANB - 1, bn)
                return 0

            lax.fori_loop(1, ABS // ANB, group, 0)
            scat_wait(ABS - 1, (ABS - 1) % ANB)
            if with_count:
                @pl.when(count_here)
                def _():
                    for b in range(ANB):
                        cnt_wait(ABS - ANB + b, (ABS - ANB + b) % ANB)

            @pl.when(bi + 1 < nb)
            def _():
                sq = lax.rem(bi + 1, 2)
                pltpu.make_async_copy(
                    src_h.at[s, pl.ds((bi + 1) * ABS, ABS)], idx_s.at[sq], isem
                ).wait()
                pltpu.make_async_copy(
                    dst_h.at[s, pl.ds((bi + 1) * ABS, ABS)], idx_d.at[sq], isem
                ).wait()
            return 0

        lax.fori_loop(0, nb, batch, 0)
        plsc.subcore_barrier()
        pltpu.sync_copy(
            acc_sh.at[pl.ds(base, rows_pt)], agg_o.at[c, pl.ds(base, rows_pt)]
        )
        if with_count:
            pltpu.sync_copy(
                cnt_sh.at[pl.ds(base, rows_pt)], cnt_o.at[c, pl.ds(base, rows_pt)]
            )

    res = run(table2, src_r, dst_r)
    if with_count:
        return res[0], res[1]
    return res[0] if isinstance(res, (tuple, list)) else res


DCH = 64


def _sc_decode(z2, la_r, lb_r, npad):
    hk = z2.shape[2]
    nch = la_r.shape[1]
    gpc = DCH // LANES
    rows_pt = npad // NS

    scratch = [
        pltpu.VMEM((nch, DCH), jnp.int32),
        pltpu.VMEM((nch, DCH), jnp.int32),
        pltpu.VMEM((nch, gpc, LANES), jnp.float32),
        pltpu.VMEM_SHARED((npad, hk, LANES), jnp.float32),
    ]
    scratch += [pltpu.VMEM((DCH, hk, LANES), jnp.float32) for _ in range(2 * DNB)]
    scratch += [pltpu.SemaphoreType.DMA] * (2 * DNB)

    @functools.partial(
        pl.kernel,
        out_type=jax.ShapeDtypeStruct((NC, NS, nch, gpc, LANES), jnp.float32),
        mesh=_mesh(),
        scratch_types=scratch,
        compiler_params=pltpu.CompilerParams(
            use_tc_tiling_on_sc=False, needs_layout_passes=False
        ),
    )
    def run(z_h, la_h, lb_h, out_h, idx_a, idx_b, obuf, z_sh, *rest):
        bufa = rest[:DNB]
        bufb = rest[DNB:2 * DNB]
        sema = rest[2 * DNB:3 * DNB]
        semb = rest[3 * DNB:4 * DNB]
        c = lax.axis_index("c")
        s = lax.axis_index("s")
        lanes = lax.iota(jnp.int32, LANES)

        base = s * rows_pt
        pltpu.sync_copy(
            z_h.at[c, pl.ds(base, rows_pt)], z_sh.at[pl.ds(base, rows_pt)]
        )
        pltpu.sync_copy(la_h.at[s], idx_a)
        pltpu.sync_copy(lb_h.at[s], idx_b)
        plsc.subcore_barrier()

        def fire(j, b):
            pltpu.async_copy(z_sh.at[idx_a.at[j]], bufa[b], sema[b])
            pltpu.async_copy(z_sh.at[idx_b.at[j]], bufb[b], semb[b])

        def wait(j, b):
            pltpu.make_async_copy(z_sh.at[idx_a.at[j]], bufa[b], sema[b]).wait()
            pltpu.make_async_copy(z_sh.at[idx_b.at[j]], bufb[b], semb[b]).wait()

        for b in range(DNB):
            fire(b, b)

        def group(g, _):
            for b in range(DNB):
                j = g * DNB + b
                wait(j, b)

                def grp16(t, _):
                    e0 = t * LANES
                    vec = jnp.zeros((LANES,), jnp.float32)
                    for ee in range(LANES):
                        acc = bufa[b][e0 + ee, 0] * bufb[b][e0 + ee, 0]
                        for k in range(1, hk):
                            acc = acc + bufa[b][e0 + ee, k] * bufb[b][e0 + ee, k]
                        vec = jnp.where(lanes == ee, jnp.sum(acc), vec)
                    obuf[j, t] = vec
                    return 0

                lax.fori_loop(0, gpc, grp16, 0)

                @pl.when(j + DNB < nch)
                def _():
                    fire(j + DNB, b)
            return 0

        lax.fori_loop(0, nch // DNB, group, 0)
        pltpu.sync_copy(obuf, out_h.at[c, s])

    return run(z2, la_r, lb_r)


def _tc_pair_sum(p):
    _, m, d = p.shape

    def body(p_r, o_r):
        o_r[...] = p_r[0] + p_r[1]

    return pl.pallas_call(
        body,
        out_shape=jax.ShapeDtypeStruct((m, d), jnp.float32),
    )(p)


def _tc_layer(agg, cnt, xin, wl, wr, b, relu, split_out):
    npad = agg.shape[1]
    dh = agg.shape[2]
    d = 2 * dh
    br = 1024
    grid = (npad // br,)
    b2 = b.reshape(1, d)

    def body(a_r, cnt_r, x_r, wl_r, wr_r, b_r, out_r):
        aggf = jnp.concatenate([a_r[0], a_r[1]], axis=1)
        xf = jnp.concatenate([x_r[0], x_r[1]], axis=1)
        cntc = jnp.clip(cnt_r[0][:, 0:1] + cnt_r[1][:, 0:1], 1.0, None)
        r = (
            jnp.dot(aggf / cntc, wl_r[...], preferred_element_type=jnp.float32)
            + jnp.dot(xf, wr_r[...], preferred_element_type=jnp.float32)
            + b_r[...]
        )
        if relu:
            r = jnp.maximum(r, 0.0)
        if split_out:
            out_r[0] = r[:, :dh]
            out_r[1] = r[:, dh:]
        else:
            out_r[...] = r

    if split_out:
        out_spec = pl.BlockSpec((NC, br, dh), lambda i: (0, i, 0))
        out_shape = jax.ShapeDtypeStruct((NC, npad, dh), jnp.float32)
    else:
        out_spec = pl.BlockSpec((br, d), lambda i: (i, 0))
        out_shape = jax.ShapeDtypeStruct((npad, d), jnp.float32)

    return pl.pallas_call(
        body,
        grid=grid,
        in_specs=[
            pl.BlockSpec((NC, br, dh), lambda i: (0, i, 0)),
            pl.BlockSpec((NC, br, LANES), lambda i: (0, i, 0)),
            pl.BlockSpec((NC, br, dh), lambda i: (0, i, 0)),
            pl.BlockSpec((d, d), lambda i: (0, 0)),
            pl.BlockSpec((d, d), lambda i: (0, 0)),
            pl.BlockSpec((1, d), lambda i: (0, 0)),
        ],
        out_specs=out_spec,
        out_shape=out_shape,
    )(agg, cnt, xin, wl, wr, b2)


def _tc_split(xin):
    npad, d = xin.shape
    dh = d // 2
    br = 1024

    def body(x_r, o_r):
        o_r[0] = x_r[:, :dh]
        o_r[1] = x_r[:, dh:]

    return pl.pallas_call(
        body,
        grid=(npad // br,),
        in_specs=[pl.BlockSpec((br, d), lambda i: (i, 0))],
        out_specs=pl.BlockSpec((NC, br, dh), lambda i: (0, i, 0)),
        out_shape=jax.ShapeDtypeStruct((NC, npad, dh), jnp.float32),
    )(xin)


def kernel(x, edge_index, edge_label_index, W1l, W1r, b1, W2l, W2r, b2):
    n, d = x.shape
    e = edge_index.shape[1]
    dh = d // 2

    npad = -(-n // 256) * 256
    if npad == n:
        npad += 256
    epa = -(-e // (NS * CH * ABS)) * (NS * CH * ABS)
    nca = epa // (NS * CH)
    epd = -(-e // (NS * DCH * DNB)) * (NS * DCH * DNB)
    ncd = epd // (NS * DCH)

    src = jnp.pad(edge_index[0], (0, epa - e)).reshape(NS, nca, CH)
    dst = jnp.pad(edge_index[1], (0, epa - e), constant_values=n).reshape(NS, nca, CH)

    la = jnp.pad(edge_label_index[0], (0, epd - e)).reshape(NS, ncd, DCH)
    lb = jnp.pad(edge_label_index[1], (0, epd - e)).reshape(NS, ncd, DCH)

    xp = jnp.pad(x, ((0, npad - n), (0, 0)))
    xsplit = _tc_split(xp)

    agg1, cnt = _sc_aggregate(xsplit, src, dst, npad, with_count=True)
    hsplit = _tc_layer(agg1, cnt, xsplit, W1l, W1r, b1, relu=True, split_out=True)
    agg2 = _sc_aggregate(hsplit, src, dst, npad, with_count=False)
    zsplit = _tc_layer(agg2, cnt, hsplit, W2l, W2r, b2, relu=False, split_out=True)
    z2 = zsplit.reshape(NC, npad, dh // LANES, LANES)
    part = _sc_decode(z2, la, lb, npad)
    p = part.reshape(NC, (NS * ncd * DCH) // d, d)
    out = _tc_pair_sum(p)
    return out.reshape(-1)[:e]
